# baseline (speedup 1.0000x reference)
import functools

import jax
import jax.numpy as jnp
from jax import lax
from jax.experimental import pallas as pl
from jax.experimental.pallas import tpu as pltpu
from jax.experimental.pallas import tpu_sc as plsc

N_NODES = 10000
NP = 10240
E = 320000
CHUNK = 128
NCH = 160
EP = 16 * NCH * CHUNK
RPT = NP // 16
BLK = 2048
NBLK = NP // BLK


def _seg_sum_colsplit(width, table2, init2, src3, dst3):
  mesh = plsc.VectorSubcoreMesh(core_axis_name="c", subcore_axis_name="s")

  @functools.partial(
      pl.kernel,
      out_type=jax.ShapeDtypeStruct((2, NP, width), jnp.float32),
      mesh=mesh,
      compiler_params=pltpu.CompilerParams(use_tc_tiling_on_sc=False),
      scratch_types=[
          pltpu.VMEM((NCH, CHUNK), jnp.int32),
          pltpu.VMEM((NCH, CHUNK), jnp.int32),
          pltpu.VMEM((2, CHUNK, width), jnp.float32),
          pltpu.VMEM_SHARED((NP, width), jnp.float32),
          pltpu.VMEM_SHARED((NP, width), jnp.float32),
          pltpu.SemaphoreType.DMA((2,)),
      ],
  )
  def seg(table_hbm, init_hbm, src_hbm, dst_hbm, out_hbm,
          src_v, dst_v, rows_v, acc_sh, table_sh, sem):
    cid = lax.axis_index("c")
    sid = lax.axis_index("s")
    pltpu.sync_copy(init_hbm.at[cid, pl.ds(sid * RPT, RPT)],
                    acc_sh.at[pl.ds(sid * RPT, RPT)])
    pltpu.sync_copy(table_hbm.at[cid, pl.ds(sid * RPT, RPT)],
                    table_sh.at[pl.ds(sid * RPT, RPT)])
    pltpu.sync_copy(src_hbm.at[sid], src_v)
    pltpu.sync_copy(dst_hbm.at[sid], dst_v)
    plsc.subcore_barrier()

    pltpu.async_copy(table_sh.at[src_v.at[0]], rows_v.at[0], sem.at[0])

    def body(j, carry):
      @pl.when(j + 1 < NCH)
      def _():
        pltpu.async_copy(table_sh.at[src_v.at[j + 1]],
                         rows_v.at[(j + 1) % 2], sem.at[(j + 1) % 2])
      pltpu.make_async_copy(table_sh.at[src_v.at[j]], rows_v.at[j % 2],
                            sem.at[j % 2]).wait()
      pltpu.sync_copy(rows_v.at[j % 2], acc_sh.at[dst_v.at[j]], add=True)
      return carry

    lax.fori_loop(0, NCH, body, 0)
    plsc.subcore_barrier()
    pltpu.sync_copy(acc_sh.at[pl.ds(sid * RPT, RPT)],
                    out_hbm.at[cid, pl.ds(sid * RPT, RPT)])

  return seg(table2, init2, src3, dst3)


def _tc1(x, W1a, W1b2):
  def body(x_ref, wa_ref, wb_ref, xa_ref, xb_ref):
    xv = x_ref[...]
    xa_ref[...] = jnp.dot(xv, wa_ref[...], preferred_element_type=jnp.float32)
    xb_ref[0] = jnp.dot(xv, wb_ref[0], preferred_element_type=jnp.float32)
    xb_ref[1] = jnp.dot(xv, wb_ref[1], preferred_element_type=jnp.float32)

  return pl.pallas_call(
      body,
      grid=(NBLK,),
      in_specs=[
          pl.BlockSpec((BLK, 128), lambda i: (i, 0)),
          pl.BlockSpec((128, 64), lambda i: (0, 0)),
          pl.BlockSpec((2, 128, 32), lambda i: (0, 0, 0)),
      ],
      out_specs=[
          pl.BlockSpec((BLK, 64), lambda i: (i, 0)),
          pl.BlockSpec((2, BLK, 32), lambda i: (0, i, 0)),
      ],
      out_shape=[
          jax.ShapeDtypeStruct((NP, 64), jnp.float32),
          jax.ShapeDtypeStruct((2, NP, 32), jnp.float32),
      ],
  )(x, W1a, W1b2)


def _tc2(xa, p, b1r, W2a2, W2b2, b2r2):
  def body(xa_ref, p_ref, b1_ref, wa_ref, wb_ref, b2_ref, ya_ref, hb_ref):
    h = xa_ref[...] + jnp.concatenate([p_ref[0], p_ref[1]], axis=1)
    h = jnp.maximum(h + b1_ref[...], 0.0)
    ya_ref[0] = jnp.dot(h, wa_ref[0],
                        preferred_element_type=jnp.float32) + b2_ref[0]
    ya_ref[1] = jnp.dot(h, wa_ref[1],
                        preferred_element_type=jnp.float32) + b2_ref[1]
    hb_ref[0] = jnp.dot(h, wb_ref[0], preferred_element_type=jnp.float32)
    hb_ref[1] = jnp.dot(h, wb_ref[1], preferred_element_type=jnp.float32)

  return pl.pallas_call(
      body,
      grid=(NBLK,),
      in_specs=[
          pl.BlockSpec((BLK, 64), lambda i: (i, 0)),
          pl.BlockSpec((2, BLK, 32), lambda i: (0, i, 0)),
          pl.BlockSpec((1, 64), lambda i: (0, 0)),
          pl.BlockSpec((2, 64, 8), lambda i: (0, 0, 0)),
          pl.BlockSpec((2, 64, 8), lambda i: (0, 0, 0)),
          pl.BlockSpec((2, 1, 8), lambda i: (0, 0, 0)),
      ],
      out_specs=[
          pl.BlockSpec((2, BLK, 8), lambda i: (0, i, 0)),
          pl.BlockSpec((2, BLK, 8), lambda i: (0, i, 0)),
      ],
      out_shape=[jax.ShapeDtypeStruct((2, NP, 8), jnp.float32)] * 2,
  )(xa, p, b1r, W2a2, W2b2, b2r2)


def kernel(x, edge_index, W1, b1, W2, b2):
  src = edge_index[0].astype(jnp.int32)
  dst = edge_index[1].astype(jnp.int32)
  pad_e = EP - E
  src_f = jnp.concatenate([src, jnp.zeros((pad_e,), jnp.int32)])
  dst_f = jnp.concatenate([dst, jnp.full((pad_e,), N_NODES, jnp.int32)])
  src16 = src_f.reshape(16, NCH, CHUNK)
  dst16 = dst_f.reshape(16, NCH, CHUNK)
  W1a = W1[:128]
  W1b2 = jnp.stack([W1[128:, :32], W1[128:, 32:]])
  W2p = jnp.pad(W2, ((0, 0), (0, 11)))
  W2a2 = jnp.stack([W2p[:64, :8], W2p[:64, 8:]])
  W2b2 = jnp.stack([W2p[64:, :8], W2p[64:, 8:]])
  b2p = jnp.pad(b2, (0, 11))
  b2r2 = jnp.stack([b2p[:8].reshape(1, 8), b2p[8:].reshape(1, 8)])
  b1r = b1.reshape(1, 64)
  zeros32 = jnp.zeros((2, NP, 32), jnp.float32)

  xa, xb2 = _tc1(x, W1a, W1b2)
  p = _seg_sum_colsplit(32, xb2, zeros32, src16, dst16)
  ya2, hb2 = _tc2(xa, p, b1r, W2a2, W2b2, b2r2)
  q = _seg_sum_colsplit(8, hb2, ya2, src16, dst16)
  return jnp.concatenate([q[0], q[1]], axis=1)[:N_NODES, :5]


# speedup vs baseline: 1.1313x; 1.1313x over previous
import functools

import jax
import jax.numpy as jnp
from jax import lax
from jax.experimental import pallas as pl
from jax.experimental.pallas import tpu as pltpu
from jax.experimental.pallas import tpu_sc as plsc

N_NODES = 10000
NP = 10240
E = 320000
CHUNK = 128
NCHT = 2500
NCH = 160
RPT = NP // 16
BLK = 2048
NBLK = NP // BLK


def _seg_sum_colsplit(width, table2, init2, idx3):
  mesh = plsc.VectorSubcoreMesh(core_axis_name="c", subcore_axis_name="s")

  @functools.partial(
      pl.kernel,
      out_type=jax.ShapeDtypeStruct((2, NP, width), jnp.float32),
      mesh=mesh,
      compiler_params=pltpu.CompilerParams(use_tc_tiling_on_sc=False),
      scratch_types=[
          pltpu.VMEM((NCH, 2, CHUNK), jnp.int32),
          pltpu.VMEM((2, CHUNK, width), jnp.float32),
          pltpu.VMEM_SHARED((NP, width), jnp.float32),
          pltpu.VMEM_SHARED((NP, width), jnp.float32),
          pltpu.SemaphoreType.DMA((2,)),
      ],
  )
  def seg(table_hbm, init_hbm, idx_hbm, out_hbm,
          sd_v, rows_v, acc_sh, table_sh, sem):
    cid = lax.axis_index("c")
    sid = lax.axis_index("s")
    base = jnp.minimum(sid * NCH, NCHT - NCH)
    skip = sid * NCH - base
    pltpu.sync_copy(init_hbm.at[cid, pl.ds(sid * RPT, RPT)],
                    acc_sh.at[pl.ds(sid * RPT, RPT)])
    pltpu.sync_copy(table_hbm.at[cid, pl.ds(sid * RPT, RPT)],
                    table_sh.at[pl.ds(sid * RPT, RPT)])
    pltpu.sync_copy(idx_hbm.at[pl.ds(base, NCH)], sd_v)
    plsc.subcore_barrier()

    pltpu.async_copy(table_sh.at[sd_v.at[skip, 0]], rows_v.at[0], sem.at[0])

    def body(j, carry):
      @pl.when(j + 1 < NCH)
      def _():
        pltpu.async_copy(table_sh.at[sd_v.at[j + 1, 0]],
                         rows_v.at[(j + 1) % 2], sem.at[(j + 1) % 2])
      pltpu.make_async_copy(table_sh.at[sd_v.at[j, 0]], rows_v.at[j % 2],
                            sem.at[j % 2]).wait()
      pltpu.sync_copy(rows_v.at[j % 2], acc_sh.at[sd_v.at[j, 1]], add=True)
      return carry

    lax.fori_loop(skip, NCH, body, 0)
    plsc.subcore_barrier()
    pltpu.sync_copy(acc_sh.at[pl.ds(sid * RPT, RPT)],
                    out_hbm.at[cid, pl.ds(sid * RPT, RPT)])

  return seg(table2, init2, idx3)


def _tc_xb(x, W1b2):
  def body(x_ref, wb_ref, xb_ref):
    xv = x_ref[...]
    xb_ref[0] = jnp.dot(xv, wb_ref[0], preferred_element_type=jnp.float32)
    xb_ref[1] = jnp.dot(xv, wb_ref[1], preferred_element_type=jnp.float32)

  return pl.pallas_call(
      body,
      grid=(NBLK,),
      in_specs=[
          pl.BlockSpec((BLK, 128), lambda i: (i, 0)),
          pl.BlockSpec((2, 128, 32), lambda i: (0, 0, 0)),
      ],
      out_specs=pl.BlockSpec((2, BLK, 32), lambda i: (0, i, 0)),
      out_shape=jax.ShapeDtypeStruct((2, NP, 32), jnp.float32),
  )(x, W1b2)


def _tc_xa(x, W1a):
  def body(x_ref, wa_ref, xa_ref):
    xa_ref[...] = jnp.dot(x_ref[...], wa_ref[...],
                          preferred_element_type=jnp.float32)

  return pl.pallas_call(
      body,
      grid=(NBLK,),
      in_specs=[
          pl.BlockSpec((BLK, 128), lambda i: (i, 0)),
          pl.BlockSpec((128, 64), lambda i: (0, 0)),
      ],
      out_specs=pl.BlockSpec((BLK, 64), lambda i: (i, 0)),
      out_shape=jax.ShapeDtypeStruct((NP, 64), jnp.float32),
  )(x, W1a)


def _tc2(xa, p, b1r, W2a2, W2b2, b2r2):
  def body(xa_ref, p_ref, b1_ref, wa_ref, wb_ref, b2_ref, ya_ref, hb_ref):
    h = xa_ref[...] + jnp.concatenate([p_ref[0], p_ref[1]], axis=1)
    h = jnp.maximum(h + b1_ref[...], 0.0)
    ya_ref[0] = jnp.dot(h, wa_ref[0],
                        preferred_element_type=jnp.float32) + b2_ref[0]
    ya_ref[1] = jnp.dot(h, wa_ref[1],
                        preferred_element_type=jnp.float32) + b2_ref[1]
    hb_ref[0] = jnp.dot(h, wb_ref[0], preferred_element_type=jnp.float32)
    hb_ref[1] = jnp.dot(h, wb_ref[1], preferred_element_type=jnp.float32)

  return pl.pallas_call(
      body,
      grid=(NBLK,),
      in_specs=[
          pl.BlockSpec((BLK, 64), lambda i: (i, 0)),
          pl.BlockSpec((2, BLK, 32), lambda i: (0, i, 0)),
          pl.BlockSpec((1, 64), lambda i: (0, 0)),
          pl.BlockSpec((2, 64, 8), lambda i: (0, 0, 0)),
          pl.BlockSpec((2, 64, 8), lambda i: (0, 0, 0)),
          pl.BlockSpec((2, 1, 8), lambda i: (0, 0, 0)),
      ],
      out_specs=[
          pl.BlockSpec((2, BLK, 8), lambda i: (0, i, 0)),
          pl.BlockSpec((2, BLK, 8), lambda i: (0, i, 0)),
      ],
      out_shape=[jax.ShapeDtypeStruct((2, NP, 8), jnp.float32)] * 2,
  )(xa, p, b1r, W2a2, W2b2, b2r2)


def kernel(x, edge_index, W1, b1, W2, b2):
  idx3 = jnp.transpose(
      edge_index.astype(jnp.int32).reshape(2, NCHT, CHUNK), (1, 0, 2))
  W1a = W1[:128]
  W1b2 = jnp.stack([W1[128:, :32], W1[128:, 32:]])
  W2p = jnp.pad(W2, ((0, 0), (0, 11)))
  W2a2 = jnp.stack([W2p[:64, :8], W2p[:64, 8:]])
  W2b2 = jnp.stack([W2p[64:, :8], W2p[64:, 8:]])
  b2p = jnp.pad(b2, (0, 11))
  b2r2 = jnp.stack([b2p[:8].reshape(1, 8), b2p[8:].reshape(1, 8)])
  b1r = b1.reshape(1, 64)
  zeros32 = jnp.zeros((2, NP, 32), jnp.float32)

  xb2 = _tc_xb(x, W1b2)
  xa = _tc_xa(x, W1a)
  p = _seg_sum_colsplit(32, xb2, zeros32, idx3)
  ya2, hb2 = _tc2(xa, p, b1r, W2a2, W2b2, b2r2)
  q = _seg_sum_colsplit(8, hb2, ya2, idx3)
  return jnp.concatenate([q[0], q[1]], axis=1)[:N_NODES, :5]


# speedup vs baseline: 1.1453x; 1.0124x over previous
import functools

import jax
import jax.numpy as jnp
from jax import lax
from jax.experimental import pallas as pl
from jax.experimental.pallas import tpu as pltpu
from jax.experimental.pallas import tpu_sc as plsc

N_NODES = 10000
NP = 10240
E = 320000
CHUNK = 128
NCHT = 2500
NCH = 160
RPT = NP // 16
BLK = 2048
NBLK = NP // BLK


def _seg_sum_colsplit(width, table2, init2, idx3):
  mesh = plsc.VectorSubcoreMesh(core_axis_name="c", subcore_axis_name="s")

  @functools.partial(
      pl.kernel,
      out_type=jax.ShapeDtypeStruct((2, NP, width), jnp.float32),
      mesh=mesh,
      compiler_params=pltpu.CompilerParams(use_tc_tiling_on_sc=False),
      scratch_types=[
          pltpu.VMEM((NCH, 2, CHUNK), jnp.int32),
          pltpu.VMEM((2, CHUNK, width), jnp.float32),
          pltpu.VMEM_SHARED((NP, width), jnp.float32),
          pltpu.VMEM_SHARED((NP, width), jnp.float32),
          pltpu.SemaphoreType.DMA((2,)),
      ],
  )
  def seg(table_hbm, init_hbm, idx_hbm, out_hbm,
          sd_v, rows_v, acc_sh, table_sh, sem):
    cid = lax.axis_index("c")
    sid = lax.axis_index("s")
    base = jnp.minimum(sid * NCH, NCHT - NCH)
    skip = sid * NCH - base
    pltpu.sync_copy(init_hbm.at[cid, pl.ds(sid * RPT, RPT)],
                    acc_sh.at[pl.ds(sid * RPT, RPT)])
    pltpu.sync_copy(table_hbm.at[cid, pl.ds(sid * RPT, RPT)],
                    table_sh.at[pl.ds(sid * RPT, RPT)])
    pltpu.sync_copy(idx_hbm.at[pl.ds(base, NCH)], sd_v)
    plsc.subcore_barrier()

    pltpu.async_copy(table_sh.at[sd_v.at[skip, 0]], rows_v.at[0], sem.at[0])

    def body(j, carry):
      @pl.when(j + 1 < NCH)
      def _():
        pltpu.async_copy(table_sh.at[sd_v.at[j + 1, 0]],
                         rows_v.at[(j + 1) % 2], sem.at[(j + 1) % 2])
      pltpu.make_async_copy(table_sh.at[sd_v.at[j, 0]], rows_v.at[j % 2],
                            sem.at[j % 2]).wait()
      pltpu.sync_copy(rows_v.at[j % 2], acc_sh.at[sd_v.at[j, 1]], add=True)
      return carry

    lax.fori_loop(skip, NCH, body, 0)
    plsc.subcore_barrier()
    pltpu.sync_copy(acc_sh.at[pl.ds(sid * RPT, RPT)],
                    out_hbm.at[cid, pl.ds(sid * RPT, RPT)])

  return seg(table2, init2, idx3)


def _seg_sum_combined(width, yh, idx3):
  mesh = plsc.VectorSubcoreMesh(core_axis_name="c", subcore_axis_name="s")

  @functools.partial(
      pl.kernel,
      out_type=jax.ShapeDtypeStruct((2, NP, width), jnp.float32),
      mesh=mesh,
      compiler_params=pltpu.CompilerParams(use_tc_tiling_on_sc=False),
      scratch_types=[
          pltpu.VMEM((NCH, 2, CHUNK), jnp.int32),
          pltpu.VMEM((2, CHUNK, width), jnp.float32),
          pltpu.VMEM_SHARED((NP, width), jnp.float32),
          pltpu.VMEM_SHARED((NP, width), jnp.float32),
          pltpu.SemaphoreType.DMA((2,)),
      ],
  )
  def seg(yh_hbm, idx_hbm, out_hbm, sd_v, rows_v, acc_sh, table_sh, sem):
    cid = lax.axis_index("c")
    sid = lax.axis_index("s")
    base = jnp.minimum(sid * NCH, NCHT - NCH)
    skip = sid * NCH - base
    pltpu.sync_copy(yh_hbm.at[cid, 0, pl.ds(sid * RPT, RPT)],
                    acc_sh.at[pl.ds(sid * RPT, RPT)])
    pltpu.sync_copy(yh_hbm.at[cid, 1, pl.ds(sid * RPT, RPT)],
                    table_sh.at[pl.ds(sid * RPT, RPT)])
    pltpu.sync_copy(idx_hbm.at[pl.ds(base, NCH)], sd_v)
    plsc.subcore_barrier()

    pltpu.async_copy(table_sh.at[sd_v.at[skip, 0]], rows_v.at[0], sem.at[0])

    def body(j, carry):
      @pl.when(j + 1 < NCH)
      def _():
        pltpu.async_copy(table_sh.at[sd_v.at[j + 1, 0]],
                         rows_v.at[(j + 1) % 2], sem.at[(j + 1) % 2])
      pltpu.make_async_copy(table_sh.at[sd_v.at[j, 0]], rows_v.at[j % 2],
                            sem.at[j % 2]).wait()
      pltpu.sync_copy(rows_v.at[j % 2], acc_sh.at[sd_v.at[j, 1]], add=True)
      return carry

    lax.fori_loop(skip, NCH, body, 0)
    plsc.subcore_barrier()
    pltpu.sync_copy(acc_sh.at[pl.ds(sid * RPT, RPT)],
                    out_hbm.at[cid, pl.ds(sid * RPT, RPT)])

  return seg(yh, idx3)


def _tc_xb(x, W1b2):
  def body(x_ref, wb_ref, xb_ref):
    xv = x_ref[...]
    xb_ref[0] = jnp.dot(xv, wb_ref[0], preferred_element_type=jnp.float32)
    xb_ref[1] = jnp.dot(xv, wb_ref[1], preferred_element_type=jnp.float32)

  return pl.pallas_call(
      body,
      grid=(NBLK,),
      in_specs=[
          pl.BlockSpec((BLK, 128), lambda i: (i, 0)),
          pl.BlockSpec((2, 128, 32), lambda i: (0, 0, 0)),
      ],
      out_specs=pl.BlockSpec((2, BLK, 32), lambda i: (0, i, 0)),
      out_shape=jax.ShapeDtypeStruct((2, NP, 32), jnp.float32),
  )(x, W1b2)


def _tc_xa(x, W1a):
  def body(x_ref, wa_ref, xa_ref):
    xa_ref[...] = jnp.dot(x_ref[...], wa_ref[...],
                          preferred_element_type=jnp.float32)

  return pl.pallas_call(
      body,
      grid=(NBLK,),
      in_specs=[
          pl.BlockSpec((BLK, 128), lambda i: (i, 0)),
          pl.BlockSpec((128, 64), lambda i: (0, 0)),
      ],
      out_specs=pl.BlockSpec((BLK, 64), lambda i: (i, 0)),
      out_shape=jax.ShapeDtypeStruct((NP, 64), jnp.float32),
  )(x, W1a)


def _tc2(xa, p, b1r, W2a2, W2b2, b2r2):
  def body(xa_ref, p_ref, b1_ref, wa_ref, wb_ref, b2_ref, yh_ref):
    h = xa_ref[...] + jnp.concatenate([p_ref[0], p_ref[1]], axis=1)
    h = jnp.maximum(h + b1_ref[...], 0.0)
    yh_ref[0, 0] = jnp.dot(h, wa_ref[0],
                           preferred_element_type=jnp.float32) + b2_ref[0]
    yh_ref[1, 0] = jnp.dot(h, wa_ref[1],
                           preferred_element_type=jnp.float32) + b2_ref[1]
    yh_ref[0, 1] = jnp.dot(h, wb_ref[0], preferred_element_type=jnp.float32)
    yh_ref[1, 1] = jnp.dot(h, wb_ref[1], preferred_element_type=jnp.float32)

  return pl.pallas_call(
      body,
      grid=(NBLK,),
      in_specs=[
          pl.BlockSpec((BLK, 64), lambda i: (i, 0)),
          pl.BlockSpec((2, BLK, 32), lambda i: (0, i, 0)),
          pl.BlockSpec((1, 64), lambda i: (0, 0)),
          pl.BlockSpec((2, 64, 8), lambda i: (0, 0, 0)),
          pl.BlockSpec((2, 64, 8), lambda i: (0, 0, 0)),
          pl.BlockSpec((2, 1, 8), lambda i: (0, 0, 0)),
      ],
      out_specs=pl.BlockSpec((2, 2, BLK, 8), lambda i: (0, 0, i, 0)),
      out_shape=jax.ShapeDtypeStruct((2, 2, NP, 8), jnp.float32),
  )(xa, p, b1r, W2a2, W2b2, b2r2)


def kernel(x, edge_index, W1, b1, W2, b2):
  idx3 = jnp.transpose(
      edge_index.astype(jnp.int32).reshape(2, NCHT, CHUNK), (1, 0, 2))
  W1a = W1[:128]
  W1b2 = jnp.stack([W1[128:, :32], W1[128:, 32:]])
  W2p = jnp.pad(W2, ((0, 0), (0, 11)))
  W2a2 = jnp.stack([W2p[:64, :8], W2p[:64, 8:]])
  W2b2 = jnp.stack([W2p[64:, :8], W2p[64:, 8:]])
  b2p = jnp.pad(b2, (0, 11))
  b2r2 = jnp.stack([b2p[:8].reshape(1, 8), b2p[8:].reshape(1, 8)])
  b1r = b1.reshape(1, 64)
  zeros32 = jnp.zeros((2, NP, 32), jnp.float32)

  xb2 = _tc_xb(x, W1b2)
  xa = _tc_xa(x, W1a)
  p = _seg_sum_colsplit(32, xb2, zeros32, idx3)
  yh = _tc2(xa, p, b1r, W2a2, W2b2, b2r2)
  q = _seg_sum_combined(8, yh, idx3)
  return q[0][:N_NODES, :5]


# speedup vs baseline: 1.1666x; 1.0186x over previous
import functools

import jax
import jax.numpy as jnp
from jax import lax
from jax.experimental import pallas as pl
from jax.experimental.pallas import tpu as pltpu
from jax.experimental.pallas import tpu_sc as plsc

N_NODES = 10000
NP = 10240
E = 320000
CHUNK = 128
NCHT = 2500
NCH = 160
RPT = NP // 16
BLK = 2048
NBLK = NP // BLK


def _seg_sum_colsplit(width, table2, init2, idx3):
  mesh = plsc.VectorSubcoreMesh(core_axis_name="c", subcore_axis_name="s")

  @functools.partial(
      pl.kernel,
      out_type=jax.ShapeDtypeStruct((2, NP, width), jnp.float32),
      mesh=mesh,
      compiler_params=pltpu.CompilerParams(use_tc_tiling_on_sc=False),
      scratch_types=[
          pltpu.VMEM((NCH, 2, CHUNK), jnp.int32),
          pltpu.VMEM((2, CHUNK, width), jnp.float32),
          pltpu.VMEM_SHARED((NP, width), jnp.float32),
          pltpu.VMEM_SHARED((NP, width), jnp.float32),
          pltpu.SemaphoreType.DMA((2,)),
      ],
  )
  def seg(table_hbm, init_hbm, idx_hbm, out_hbm,
          sd_v, rows_v, acc_sh, table_sh, sem):
    cid = lax.axis_index("c")
    sid = lax.axis_index("s")
    base = jnp.minimum(sid * NCH, NCHT - NCH)
    skip = sid * NCH - base
    pltpu.sync_copy(init_hbm.at[cid, pl.ds(sid * RPT, RPT)],
                    acc_sh.at[pl.ds(sid * RPT, RPT)])
    pltpu.sync_copy(table_hbm.at[cid, pl.ds(sid * RPT, RPT)],
                    table_sh.at[pl.ds(sid * RPT, RPT)])
    pltpu.sync_copy(idx_hbm.at[pl.ds(base, NCH)], sd_v)
    plsc.subcore_barrier()

    pltpu.async_copy(table_sh.at[sd_v.at[skip, 0]], rows_v.at[0], sem.at[0])

    def body(j, carry):
      @pl.when(j + 1 < NCH)
      def _():
        pltpu.async_copy(table_sh.at[sd_v.at[j + 1, 0]],
                         rows_v.at[(j + 1) % 2], sem.at[(j + 1) % 2])
      pltpu.make_async_copy(table_sh.at[sd_v.at[j, 0]], rows_v.at[j % 2],
                            sem.at[j % 2]).wait()
      pltpu.sync_copy(rows_v.at[j % 2], acc_sh.at[sd_v.at[j, 1]], add=True)
      return carry

    lax.fori_loop(skip, NCH, body, 0)
    plsc.subcore_barrier()
    pltpu.sync_copy(acc_sh.at[pl.ds(sid * RPT, RPT)],
                    out_hbm.at[cid, pl.ds(sid * RPT, RPT)])

  return seg(table2, init2, idx3)


def _seg_sum_combined(width, yh, idx3):
  mesh = plsc.VectorSubcoreMesh(core_axis_name="c", subcore_axis_name="s")

  @functools.partial(
      pl.kernel,
      out_type=jax.ShapeDtypeStruct((NP, width), jnp.float32),
      mesh=mesh,
      compiler_params=pltpu.CompilerParams(use_tc_tiling_on_sc=False),
      scratch_types=[
          pltpu.VMEM((NCH, 2, CHUNK), jnp.int32),
          pltpu.VMEM((2, CHUNK, width), jnp.float32),
          pltpu.VMEM_SHARED((NP, width), jnp.float32),
          pltpu.VMEM_SHARED((NP, width), jnp.float32),
          pltpu.SemaphoreType.DMA((2,)),
      ],
  )
  def seg(yh_hbm, idx_hbm, out_hbm, sd_v, rows_v, acc_sh, table_sh, sem):
    cid = lax.axis_index("c")
    sid = lax.axis_index("s")
    base = jnp.minimum(sid * NCH, NCHT - NCH)
    skip = sid * NCH - base
    pltpu.sync_copy(yh_hbm.at[cid, 0, pl.ds(sid * RPT, RPT)],
                    acc_sh.at[pl.ds(sid * RPT, RPT)])
    pltpu.sync_copy(yh_hbm.at[cid, 1, pl.ds(sid * RPT, RPT)],
                    table_sh.at[pl.ds(sid * RPT, RPT)])
    pltpu.sync_copy(idx_hbm.at[pl.ds(base, NCH)], sd_v)
    plsc.subcore_barrier()

    pltpu.async_copy(table_sh.at[sd_v.at[skip, 0]], rows_v.at[0], sem.at[0])

    def body(j, carry):
      @pl.when(j + 1 < NCH)
      def _():
        pltpu.async_copy(table_sh.at[sd_v.at[j + 1, 0]],
                         rows_v.at[(j + 1) % 2], sem.at[(j + 1) % 2])
      pltpu.make_async_copy(table_sh.at[sd_v.at[j, 0]], rows_v.at[j % 2],
                            sem.at[j % 2]).wait()
      pltpu.sync_copy(rows_v.at[j % 2], acc_sh.at[sd_v.at[j, 1]], add=True)
      return carry

    lax.fori_loop(skip, NCH, body, 0)
    plsc.subcore_barrier()

    @pl.when(cid == 0)
    def _():
      pltpu.sync_copy(acc_sh.at[pl.ds(sid * RPT, RPT)],
                      out_hbm.at[pl.ds(sid * RPT, RPT)])

  return seg(yh, idx3)


def _tc_xb(x, W1b2):
  def body(x_ref, wb_ref, xb_ref):
    xv = x_ref[...]
    xb_ref[0] = jnp.dot(xv, wb_ref[0], preferred_element_type=jnp.float32)
    xb_ref[1] = jnp.dot(xv, wb_ref[1], preferred_element_type=jnp.float32)

  return pl.pallas_call(
      body,
      grid=(NBLK,),
      in_specs=[
          pl.BlockSpec((BLK, 128), lambda i: (i, 0)),
          pl.BlockSpec((2, 128, 32), lambda i: (0, 0, 0)),
      ],
      out_specs=pl.BlockSpec((2, BLK, 32), lambda i: (0, i, 0)),
      out_shape=jax.ShapeDtypeStruct((2, NP, 32), jnp.float32),
  )(x, W1b2)


def _tc_xa(x, W1a):
  def body(x_ref, wa_ref, xa_ref):
    xa_ref[...] = jnp.dot(x_ref[...], wa_ref[...],
                          preferred_element_type=jnp.float32)

  return pl.pallas_call(
      body,
      grid=(NBLK,),
      in_specs=[
          pl.BlockSpec((BLK, 128), lambda i: (i, 0)),
          pl.BlockSpec((128, 64), lambda i: (0, 0)),
      ],
      out_specs=pl.BlockSpec((BLK, 64), lambda i: (i, 0)),
      out_shape=jax.ShapeDtypeStruct((NP, 64), jnp.float32),
  )(x, W1a)


def _tc2(xa, p, b1r, W2a2, W2b2, b2r2):
  def body(xa_ref, p_ref, b1_ref, wa_ref, wb_ref, b2_ref, yh_ref):
    h = xa_ref[...] + jnp.concatenate([p_ref[0], p_ref[1]], axis=1)
    h = jnp.maximum(h + b1_ref[...], 0.0)
    yh_ref[0, 0] = jnp.dot(h, wa_ref[0],
                           preferred_element_type=jnp.float32) + b2_ref[0]
    yh_ref[1, 0] = jnp.dot(h, wa_ref[1],
                           preferred_element_type=jnp.float32) + b2_ref[1]
    yh_ref[0, 1] = jnp.dot(h, wb_ref[0], preferred_element_type=jnp.float32)
    yh_ref[1, 1] = jnp.dot(h, wb_ref[1], preferred_element_type=jnp.float32)

  return pl.pallas_call(
      body,
      grid=(NBLK,),
      in_specs=[
          pl.BlockSpec((BLK, 64), lambda i: (i, 0)),
          pl.BlockSpec((2, BLK, 32), lambda i: (0, i, 0)),
          pl.BlockSpec((1, 64), lambda i: (0, 0)),
          pl.BlockSpec((2, 64, 8), lambda i: (0, 0, 0)),
          pl.BlockSpec((2, 64, 8), lambda i: (0, 0, 0)),
          pl.BlockSpec((2, 1, 8), lambda i: (0, 0, 0)),
      ],
      out_specs=pl.BlockSpec((2, 2, BLK, 8), lambda i: (0, 0, i, 0)),
      out_shape=jax.ShapeDtypeStruct((2, 2, NP, 8), jnp.float32),
  )(xa, p, b1r, W2a2, W2b2, b2r2)


def kernel(x, edge_index, W1, b1, W2, b2):
  idx3 = jnp.transpose(
      edge_index.astype(jnp.int32).reshape(2, NCHT, CHUNK), (1, 0, 2))
  W1a = W1[:128]
  W1b2 = jnp.stack([W1[128:, :32], W1[128:, 32:]])
  W2p = jnp.pad(W2, ((0, 0), (0, 11)))
  W2a2 = jnp.stack([W2p[:64, :8], W2p[:64, 8:]])
  W2b2 = jnp.stack([W2p[64:, :8], W2p[64:, 8:]])
  b2p = jnp.pad(b2, (0, 11))
  b2r2 = jnp.stack([b2p[:8].reshape(1, 8), b2p[8:].reshape(1, 8)])
  b1r = b1.reshape(1, 64)
  zeros32 = jnp.zeros((2, NP, 32), jnp.float32)

  xb2 = _tc_xb(x, W1b2)
  xa = _tc_xa(x, W1a)
  p = _seg_sum_colsplit(32, xb2, zeros32, idx3)
  yh = _tc2(xa, p, b1r, W2a2, W2b2, b2r2)
  q = _seg_sum_combined(8, yh, idx3)
  return q[:N_NODES, :5]
